# baseline (device time: 20933 ns/iter reference)
import os

import jax
import jax.numpy as jnp
from jax import lax
from jax.experimental import pallas as pl
from jax.experimental.pallas import tpu as pltpu

WORLD = 32
N_TOK = 512
D_IN = 256
D_OUT = 512
E_LOCAL = 4
CAP = 3
CAP_PAD = 4
SLOT = E_LOCAL * CAP_PAD
ROWS = N_TOK // WORLD

_PROBE = os.environ.get("KERNEL_PROBE", "full")


def kernel(x, router_W, route_idx, expert_W):
    def body(x_ref, rW_ref, idx_ref, eW_ref, out_ref,
             y_ref, gather_ref, send_sems, recv_sems):
        d = lax.axis_index("i")

        if _PROBE != "compute":
            barrier_sem = pltpu.get_barrier_semaphore()
            for t in range(1, WORLD):
                pl.semaphore_signal(
                    barrier_sem, inc=1,
                    device_id=((d + t) % WORLD,),
                    device_id_type=pl.DeviceIdType.MESH,
                )

        idx_v = idx_ref[:, 0]
        ti = lax.broadcasted_iota(jnp.int32, (N_TOK, N_TOK), 0)
        tj = lax.broadcasted_iota(jnp.int32, (N_TOK, N_TOK), 1)
        lower = (tj < ti).astype(jnp.float32)
        e_iota = lax.broadcasted_iota(jnp.int32, (N_TOK, E_LOCAL), 1)
        onehot = (idx_v[:, None] == d * E_LOCAL + e_iota).astype(jnp.float32)
        ranks = jnp.dot(lower, onehot)
        rank_all = jnp.sum(onehot * ranks, axis=1)

        r_iota = lax.broadcasted_iota(jnp.int32, (SLOT, N_TOK), 0)
        r_e = r_iota // CAP
        r_k = r_iota % CAP
        disp = (
            (idx_v[None, :] == d * E_LOCAL + r_e)
            & (rank_all[None, :] == r_k.astype(jnp.float32))
            & (r_iota < E_LOCAL * CAP)
        ).astype(jnp.float32)

        xg = jnp.dot(disp, x_ref[:, :])
        r_row = lax.broadcasted_iota(jnp.int32, (SLOT, 1), 0) // CAP
        y = jnp.zeros((SLOT, D_OUT), jnp.float32)
        for e in range(E_LOCAL):
            xe = jnp.where(r_row == e, xg, 0.0)
            y = y + jnp.dot(xe, eW_ref[e])
        y_ref[:, :] = y.astype(jnp.bfloat16)

        rdmas = []
        if _PROBE != "compute":
            pl.semaphore_wait(barrier_sem, WORLD - 1)
            for t in range(1, WORLD):
                dst = (d + t) % WORLD
                rdma = pltpu.make_async_remote_copy(
                    src_ref=y_ref,
                    dst_ref=gather_ref.at[
                        pl.ds((t - 1) * SLOT, SLOT), :
                    ],
                    send_sem=send_sems.at[t - 1],
                    recv_sem=recv_sems.at[t - 1],
                    device_id=(dst,),
                    device_id_type=pl.DeviceIdType.MESH,
                )
                rdma.start()
                rdmas.append(rdma)
        gather_ref[(WORLD - 1) * SLOT:, :] = y.astype(jnp.bfloat16)

        idx_loc = idx_ref[pl.ds(d * ROWS, ROWS), 0]
        a_tok = d * ROWS + lax.broadcasted_iota(jnp.int32, (ROWS, N_TOK), 0)
        j512 = lax.broadcasted_iota(jnp.int32, (ROWS, N_TOK), 1)
        cmp = (idx_loc[:, None] == idx_v[None, :]) & (j512 < a_tok)
        rank_loc = jnp.sum(cmp.astype(jnp.float32), axis=1)

        jj = lax.broadcasted_iota(jnp.int32, (ROWS, WORLD * SLOT), 1)
        u_j = jj // SLOT
        r_j = jj % SLOT
        src_j = jnp.where(u_j < WORLD - 1, (d + WORLD - 1 - u_j) % WORLD, d)
        e_glob = src_j * E_LOCAL + r_j // CAP
        k_j = (r_j % CAP).astype(jnp.float32)
        dmat = (
            (idx_loc[:, None] == e_glob)
            & (rank_loc[:, None] == k_j)
            & (r_j < E_LOCAL * CAP)
        ).astype(jnp.bfloat16)

        if _PROBE != "compute":
            for u in range(1, WORLD):
                rdmas[u - 1].wait_recv()
        out_ref[:, :] = jnp.dot(
            dmat, gather_ref[:, :], preferred_element_type=jnp.float32
        )

        if _PROBE != "compute":
            for t in range(1, WORLD):
                rdmas[t - 1].wait_send()

    return pl.pallas_call(
        body,
        out_shape=jax.ShapeDtypeStruct((ROWS, D_OUT), jnp.float32),
        in_specs=[pl.BlockSpec(memory_space=pltpu.VMEM)] * 4,
        out_specs=pl.BlockSpec(memory_space=pltpu.VMEM),
        scratch_shapes=[
            pltpu.VMEM((SLOT, D_OUT), jnp.bfloat16),
            pltpu.VMEM((WORLD * SLOT, D_OUT), jnp.bfloat16),
            pltpu.SemaphoreType.DMA((WORLD - 1,)),
            pltpu.SemaphoreType.DMA((WORLD - 1,)),
        ],
        compiler_params=(
            None
            if _PROBE == "compute"
            else pltpu.CompilerParams(collective_id=0)
        ),
    )(x, router_W, route_idx, expert_W)


# device time: 19991 ns/iter; 1.0471x vs baseline; 1.0471x over previous
import os

import jax
import jax.numpy as jnp
from jax import lax
from jax.experimental import pallas as pl
from jax.experimental.pallas import tpu as pltpu

WORLD = 32
N_TOK = 512
D_IN = 256
D_OUT = 512
E_LOCAL = 4
CAP = 3
CAP_PAD = 4
SLOT = E_LOCAL * CAP_PAD
ROWS = N_TOK // WORLD

_PROBE = os.environ.get("KERNEL_PROBE", "full")


def kernel(x, router_W, route_idx, expert_W):
    def body(x_ref, idx_ref, eW_ref, out_ref,
             y_ref, gather_ref, send_sems, recv_sems):
        d = lax.axis_index("i")

        if _PROBE != "compute":
            barrier_sem = pltpu.get_barrier_semaphore()
            for t in range(1, WORLD):
                pl.semaphore_signal(
                    barrier_sem, inc=1,
                    device_id=((d + t) % WORLD,),
                    device_id_type=pl.DeviceIdType.MESH,
                )

        idx_v = idx_ref[:, 0]
        ti = lax.broadcasted_iota(jnp.int32, (N_TOK, N_TOK), 0)
        tj = lax.broadcasted_iota(jnp.int32, (N_TOK, N_TOK), 1)
        lower = (tj < ti).astype(jnp.float32)
        e_iota = lax.broadcasted_iota(jnp.int32, (N_TOK, E_LOCAL), 1)
        onehot = (idx_v[:, None] == d * E_LOCAL + e_iota).astype(jnp.float32)
        ranks = jnp.dot(lower, onehot)
        rank_all = jnp.sum(onehot * ranks, axis=1)

        r_iota = lax.broadcasted_iota(jnp.int32, (SLOT, N_TOK), 0)
        r_e = r_iota // CAP
        r_k = r_iota % CAP
        disp = (
            (idx_v[None, :] == d * E_LOCAL + r_e)
            & (rank_all[None, :] == r_k.astype(jnp.float32))
            & (r_iota < E_LOCAL * CAP)
        ).astype(jnp.float32)

        xg = jnp.dot(disp, x_ref[:, :])
        r_row = lax.broadcasted_iota(jnp.int32, (SLOT, 1), 0) // CAP
        y = jnp.zeros((SLOT, D_OUT), jnp.float32)
        for e in range(E_LOCAL):
            xe = jnp.where(r_row == e, xg, 0.0)
            y = y + jnp.dot(xe, eW_ref[e])
        y_ref[:, :] = y.astype(jnp.bfloat16)

        rdmas = []
        if _PROBE != "compute":
            pl.semaphore_wait(barrier_sem, WORLD - 1)
            for t in range(1, WORLD):
                dst = (d + t) % WORLD
                rdma = pltpu.make_async_remote_copy(
                    src_ref=y_ref,
                    dst_ref=gather_ref.at[
                        pl.ds((t - 1) * SLOT, SLOT), :
                    ],
                    send_sem=send_sems.at[t - 1],
                    recv_sem=recv_sems.at[t - 1],
                    device_id=(dst,),
                    device_id_type=pl.DeviceIdType.MESH,
                )
                rdma.start()
                rdmas.append(rdma)
        gather_ref[(WORLD - 1) * SLOT:, :] = y.astype(jnp.bfloat16)

        idx_loc = idx_ref[pl.ds(d * ROWS, ROWS), 0]
        a_tok = d * ROWS + lax.broadcasted_iota(jnp.int32, (ROWS, N_TOK), 0)
        j512 = lax.broadcasted_iota(jnp.int32, (ROWS, N_TOK), 1)
        cmp = (idx_loc[:, None] == idx_v[None, :]) & (j512 < a_tok)
        rank_loc = jnp.sum(cmp.astype(jnp.float32), axis=1)

        jj = lax.broadcasted_iota(jnp.int32, (ROWS, WORLD * SLOT), 1)
        u_j = jj // SLOT
        r_j = jj % SLOT
        src_j = jnp.where(u_j < WORLD - 1, (d + WORLD - 1 - u_j) % WORLD, d)
        e_glob = src_j * E_LOCAL + r_j // CAP
        k_j = (r_j % CAP).astype(jnp.float32)
        dmat = (
            (idx_loc[:, None] == e_glob)
            & (rank_loc[:, None] == k_j)
            & (r_j < E_LOCAL * CAP)
        ).astype(jnp.bfloat16)

        if _PROBE != "compute":
            for u in range(1, WORLD):
                rdmas[u - 1].wait_recv()
        out_ref[:, :] = jnp.dot(
            dmat, gather_ref[:, :], preferred_element_type=jnp.float32
        )

        if _PROBE != "compute":
            for t in range(1, WORLD):
                rdmas[t - 1].wait_send()

    return pl.pallas_call(
        body,
        out_shape=jax.ShapeDtypeStruct((ROWS, D_OUT), jnp.float32),
        in_specs=[pl.BlockSpec(memory_space=pltpu.VMEM)] * 3,
        out_specs=pl.BlockSpec(memory_space=pltpu.VMEM),
        scratch_shapes=[
            pltpu.VMEM((SLOT, D_OUT), jnp.bfloat16),
            pltpu.VMEM((WORLD * SLOT, D_OUT), jnp.bfloat16),
            pltpu.SemaphoreType.DMA((WORLD - 1,)),
            pltpu.SemaphoreType.DMA((WORLD - 1,)),
        ],
        compiler_params=(
            None
            if _PROBE == "compute"
            else pltpu.CompilerParams(collective_id=0)
        ),
    )(x, route_idx, expert_W)
